# baseline (speedup 1.0000x reference)
import jax
import jax.numpy as jnp
from jax import lax
from jax.experimental import pallas as pl
from jax.experimental.pallas import tpu as pltpu

N_BASES = 4
L = 2048
BLOCK = 64
NBUF = 6
NSTEP = L // BLOCK


def _half(vec128, i):
    lo = vec128[:BLOCK]
    hi = vec128[BLOCK:]
    return jnp.where(lax.rem(i, 2) == 0, lo, hi)


def _body(seq_ref, pair_ref, seq_hbm, idx_hbm,
          seq_buf, idx_buf, seq_sem, idx_sem):
    i = pl.program_id(0)
    slot = lax.rem(i, NBUF)

    def _seq_copy(step, s):
        return pltpu.make_async_copy(
            seq_buf.at[s],
            seq_hbm.at[:, pl.ds(step * BLOCK, BLOCK), :],
            seq_sem.at[s])

    def _idx_copy(step, s):
        return pltpu.make_async_copy(
            idx_buf.at[s],
            idx_hbm.at[:, pl.ds(step * BLOCK, BLOCK), :],
            idx_sem.at[s])

    @pl.when(i >= NBUF)
    def _():
        _seq_copy(i - NBUF, slot).wait()
        _idx_copy(i - NBUF, slot).wait()

    base = lax.div(i, 2) * (2 * BLOCK)
    si = _half(seq_ref[0, pl.ds(base, 2 * BLOCK)], i)
    sj = seq_ref[0, :]
    pi = _half(pair_ref[0, pl.ds(base, 2 * BLOCK)], i)
    jj = lax.broadcasted_iota(jnp.int32, (BLOCK, L), 1)
    for c in range(N_BASES):
        seq_buf[slot, c] = jnp.broadcast_to(
            (si[:, None] == c).astype(jnp.float32), (BLOCK, L))
    for c in range(N_BASES):
        seq_buf[slot, c + N_BASES] = jnp.broadcast_to(
            (sj[None, :] == c).astype(jnp.float32), (BLOCK, L))
    idx_buf[slot, 0] = (pi[:, None] == jj).astype(jnp.float32)

    _seq_copy(i, slot).start()
    _idx_copy(i, slot).start()

    @pl.when(i == NSTEP - 1)
    def _():
        for d in range(min(NBUF, NSTEP)):
            s = lax.rem(i - d + NBUF, NBUF)
            _seq_copy(i - d, s).wait()
            _idx_copy(i - d, s).wait()


def kernel(seq_idx, pair_idx):
    n = seq_idx.shape[0]
    seq_out, idx_out = pl.pallas_call(
        _body,
        grid=(NSTEP,),
        in_specs=[
            pl.BlockSpec((1, n), lambda i: (0, 0)),
            pl.BlockSpec((1, n), lambda i: (0, 0)),
        ],
        out_specs=[
            pl.BlockSpec(memory_space=pl.ANY),
            pl.BlockSpec(memory_space=pl.ANY),
        ],
        out_shape=[
            jax.ShapeDtypeStruct((2 * N_BASES, n, n), jnp.float32),
            jax.ShapeDtypeStruct((1, n, n), jnp.float32),
        ],
        scratch_shapes=[
            pltpu.VMEM((NBUF, 2 * N_BASES, BLOCK, L), jnp.float32),
            pltpu.VMEM((NBUF, 1, BLOCK, L), jnp.float32),
            pltpu.SemaphoreType.DMA((NBUF,)),
            pltpu.SemaphoreType.DMA((NBUF,)),
        ],
        compiler_params=pltpu.CompilerParams(
            dimension_semantics=("arbitrary",)),
    )(seq_idx.reshape(1, n), pair_idx.reshape(1, n))
    return (seq_out, idx_out)


# speedup vs baseline: 1.0053x; 1.0053x over previous
import jax
import jax.numpy as jnp
from jax import lax
from jax.experimental import pallas as pl
from jax.experimental.pallas import tpu as pltpu

N_BASES = 4
L = 2048
BLOCK = 64
NBUF = 4
NSTEP = L // BLOCK


def _half(vec128, i):
    lo = vec128[:BLOCK]
    hi = vec128[BLOCK:]
    return jnp.where(lax.rem(i, 2) == 0, lo, hi)


def _body(seq_ref, pair_ref, seq_hbm, idx_hbm,
          seq_buf, idx_buf, seq_sem, idx_sem):
    i = pl.program_id(0)
    slot = lax.rem(i, NBUF)

    def _seq_copies(step, s):
        return [pltpu.make_async_copy(
            seq_buf.at[s, c],
            seq_hbm.at[c, pl.ds(step * BLOCK, BLOCK), :],
            seq_sem.at[s]) for c in range(2 * N_BASES)]

    def _idx_copy(step, s):
        return pltpu.make_async_copy(
            idx_buf.at[s],
            idx_hbm.at[:, pl.ds(step * BLOCK, BLOCK), :],
            idx_sem.at[s])

    @pl.when(i >= NBUF)
    def _():
        for cp in _seq_copies(i - NBUF, slot):
            cp.wait()
        _idx_copy(i - NBUF, slot).wait()

    base = lax.div(i, 2) * (2 * BLOCK)
    si = _half(seq_ref[0, pl.ds(base, 2 * BLOCK)], i)
    sj = seq_ref[0, :]
    pi = _half(pair_ref[0, pl.ds(base, 2 * BLOCK)], i)
    jj = lax.broadcasted_iota(jnp.int32, (BLOCK, L), 1)
    for c in range(N_BASES):
        seq_buf[slot, c] = jnp.broadcast_to(
            (si[:, None] == c).astype(jnp.float32), (BLOCK, L))
    for c in range(N_BASES):
        seq_buf[slot, c + N_BASES] = jnp.broadcast_to(
            (sj[None, :] == c).astype(jnp.float32), (BLOCK, L))
    idx_buf[slot, 0] = (pi[:, None] == jj).astype(jnp.float32)

    for cp in _seq_copies(i, slot):
        cp.start()
    _idx_copy(i, slot).start()

    @pl.when(i == NSTEP - 1)
    def _():
        for d in range(min(NBUF, NSTEP)):
            s = lax.rem(i - d + NBUF, NBUF)
            for cp in _seq_copies(i - d, s):
                cp.wait()
            _idx_copy(i - d, s).wait()


def kernel(seq_idx, pair_idx):
    n = seq_idx.shape[0]
    seq_out, idx_out = pl.pallas_call(
        _body,
        grid=(NSTEP,),
        in_specs=[
            pl.BlockSpec((1, n), lambda i: (0, 0)),
            pl.BlockSpec((1, n), lambda i: (0, 0)),
        ],
        out_specs=[
            pl.BlockSpec(memory_space=pl.ANY),
            pl.BlockSpec(memory_space=pl.ANY),
        ],
        out_shape=[
            jax.ShapeDtypeStruct((2 * N_BASES, n, n), jnp.float32),
            jax.ShapeDtypeStruct((1, n, n), jnp.float32),
        ],
        scratch_shapes=[
            pltpu.VMEM((NBUF, 2 * N_BASES, BLOCK, L), jnp.float32),
            pltpu.VMEM((NBUF, 1, BLOCK, L), jnp.float32),
            pltpu.SemaphoreType.DMA((NBUF,)),
            pltpu.SemaphoreType.DMA((NBUF,)),
        ],
        compiler_params=pltpu.CompilerParams(
            dimension_semantics=("arbitrary",)),
    )(seq_idx.reshape(1, n), pair_idx.reshape(1, n))
    return (seq_out, idx_out)
